# baseline (speedup 1.0000x reference)
import functools

import jax
import jax.numpy as jnp
from jax import lax
from jax.experimental import pallas as pl
from jax.experimental.pallas import tpu as pltpu
from jax.experimental.pallas import tpu_sc as plsc

WALK = 10
E = 128
B = 16384
NC = 2
NS = 16
NW = NC * NS
C = 128
K = 4

IN_CH = B // (NW * C)
POS_CH = B * WALK // (NW * C)
NEG_CH = B * WALK * 5 // (NW * C)

_MESH = plsc.VectorSubcoreMesh(core_axis_name="c", subcore_axis_name="s")


def _gin_body(in_idx, w_in, o_emb, in_v, b0, b1, b2, b3, g0, g1, g2, g3,
              s0, s1, s2, s3):
    bufs = (b0, b1, b2, b3)
    gsems = (g0, g1, g2, g3)
    ssems = (s0, s1, s2, s3)
    wid = lax.axis_index("s") * NC + lax.axis_index("c")
    pltpu.sync_copy(in_idx.at[wid], in_v)
    base = wid * (B // NW)
    gh = [pltpu.async_copy(w_in.at[in_v.at[j]], bufs[j], gsems[j])
          for j in range(IN_CH)]
    sh = []
    for j in range(IN_CH):
        gh[j].wait()
        sh.append(pltpu.async_copy(bufs[j], o_emb.at[pl.ds(base + j * C, C)],
                                   ssems[j]))
    for h in sh:
        h.wait()


_gather_in = functools.partial(
    pl.kernel,
    mesh=_MESH,
    out_type=jax.ShapeDtypeStruct((B, E), jnp.float32),
    scratch_types=[
        pltpu.VMEM((IN_CH, C), jnp.int32),
    ] + [pltpu.VMEM((C, E), jnp.float32) for _ in range(K)]
      + [pltpu.SemaphoreType.DMA for _ in range(2 * K)],
)(_gin_body)


def _posneg_body(pos_idx, neg_idx, w_out, o_pos, o_neg, w_out_sh,
                 pos_v, neg_v, b0, b1, b2, b3, g0, g1, g2, g3,
                 s0, s1, s2, s3):
    bufs = (b0, b1, b2, b3)
    gsems = (g0, g1, g2, g3)
    ssems = (s0, s1, s2, s3)
    sid = lax.axis_index("s")
    wid = sid * NC + lax.axis_index("c")

    @pl.when(sid == 0)
    def _():
        pltpu.sync_copy(w_out, w_out_sh)

    pltpu.sync_copy(pos_idx.at[wid], pos_v)
    pltpu.sync_copy(neg_idx.at[wid], neg_v)
    plsc.subcore_barrier()

    def drain_scatter(b, out):
        pltpu.make_async_copy(bufs[b], out.at[pl.ds(0, C)], ssems[b]).wait()

    def run_phase(idx_v, out, nch, base_row):
        ngrp = nch // K

        def group(i, carry):
            gh = []
            for b in range(K):
                @pl.when(i != 0)
                def _(b=b):
                    drain_scatter(b, out)
                gh.append(pltpu.async_copy(
                    w_out_sh.at[idx_v.at[i * K + b]], bufs[b], gsems[b]))
            for b in range(K):
                gh[b].wait()
                row0 = base_row + (i * K + b) * C
                pltpu.async_copy(bufs[b], out.at[pl.ds(row0, C)], ssems[b])
            return carry

        lax.fori_loop(0, ngrp, group, 0)
        for b in range(K):
            drain_scatter(b, out)

    run_phase(pos_v, o_pos, POS_CH, wid * POS_CH * C)
    run_phase(neg_v, o_neg, NEG_CH, wid * NEG_CH * C)


_posneg = functools.partial(
    pl.kernel,
    mesh=_MESH,
    out_type=(
        jax.ShapeDtypeStruct((B * WALK, E), jnp.float32),
        jax.ShapeDtypeStruct((B * WALK * 5, E), jnp.float32),
    ),
    scratch_types=[
        pltpu.VMEM_SHARED((2405, E), jnp.float32),
        pltpu.VMEM((POS_CH, C), jnp.int32),
        pltpu.VMEM((NEG_CH, C), jnp.int32),
    ] + [pltpu.VMEM((C, E), jnp.float32) for _ in range(K)]
      + [pltpu.SemaphoreType.DMA for _ in range(2 * K)],
)(_posneg_body)


TBLK = 2048


def _tile_body(in_ref, out_ref):
    out_ref[...] = in_ref[...]


_tile = pl.pallas_call(
    _tile_body,
    grid=(B // TBLK, WALK),
    in_specs=[pl.BlockSpec((TBLK, E), lambda j, k: (j, 0))],
    out_specs=pl.BlockSpec((TBLK, E), lambda j, k: (k * (B // TBLK) + j, 0)),
    out_shape=jax.ShapeDtypeStruct((B * WALK, E), jnp.float32),
)


def kernel(input_labels, pos_labels, neg_labels, W_in, W_out):
    in_idx = input_labels.reshape(NW, IN_CH, C).astype(jnp.int32)
    pos_idx = pos_labels.reshape(NW, POS_CH, C).astype(jnp.int32)
    neg_idx = neg_labels.reshape(NW, NEG_CH, C).astype(jnp.int32)
    out_pos, out_neg = _posneg(pos_idx, neg_idx, W_out)
    in_emb = _gather_in(in_idx, W_in)
    out_in = _tile(in_emb)
    return out_in, out_pos, out_neg


# speedup vs baseline: 1.0184x; 1.0184x over previous
import functools

import jax
import jax.numpy as jnp
from jax import lax
from jax.experimental import pallas as pl
from jax.experimental.pallas import tpu as pltpu
from jax.experimental.pallas import tpu_sc as plsc

WALK = 10
E = 128
B = 16384
NC = 2
NS = 16
NW = NC * NS
C = 128
K = 4

IN_CH = B // (NW * C)
POS_CH = B * WALK // (NW * C)
NEG_CH = B * WALK * 5 // (NW * C)

_MESH = plsc.VectorSubcoreMesh(core_axis_name="c", subcore_axis_name="s")


def _posneg_body(pos_idx, neg_idx, w_out, o_pos, o_neg, w_out_sh,
                 pos_v, neg_v, b0, b1, b2, b3, g0, g1, g2, g3,
                 s0, s1, s2, s3):
    bufs = (b0, b1, b2, b3)
    gsems = (g0, g1, g2, g3)
    ssems = (s0, s1, s2, s3)
    sid = lax.axis_index("s")
    wid = sid * NC + lax.axis_index("c")

    @pl.when(sid == 0)
    def _():
        pltpu.sync_copy(w_out, w_out_sh)

    pltpu.sync_copy(pos_idx.at[wid], pos_v)
    pltpu.sync_copy(neg_idx.at[wid], neg_v)
    plsc.subcore_barrier()

    def drain_scatter(b, out):
        pltpu.make_async_copy(bufs[b], out.at[pl.ds(0, C)], ssems[b]).wait()

    def run_phase(idx_v, out, nch, base_row):
        ngrp = nch // K

        def group(i, carry):
            gh = []
            for b in range(K):
                @pl.when(i != 0)
                def _(b=b):
                    drain_scatter(b, out)
                gh.append(pltpu.async_copy(
                    w_out_sh.at[idx_v.at[i * K + b]], bufs[b], gsems[b]))
            for b in range(K):
                gh[b].wait()
                row0 = base_row + (i * K + b) * C
                pltpu.async_copy(bufs[b], out.at[pl.ds(row0, C)], ssems[b])
            return carry

        lax.fori_loop(0, ngrp, group, 0)
        for b in range(K):
            drain_scatter(b, out)

    run_phase(pos_v, o_pos, POS_CH, wid * POS_CH * C)
    run_phase(neg_v, o_neg, NEG_CH, wid * NEG_CH * C)


_posneg = functools.partial(
    pl.kernel,
    mesh=_MESH,
    out_type=(
        jax.ShapeDtypeStruct((B * WALK, E), jnp.float32),
        jax.ShapeDtypeStruct((B * WALK * 5, E), jnp.float32),
    ),
    scratch_types=[
        pltpu.VMEM_SHARED((2405, E), jnp.float32),
        pltpu.VMEM((POS_CH, C), jnp.int32),
        pltpu.VMEM((NEG_CH, C), jnp.int32),
    ] + [pltpu.VMEM((C, E), jnp.float32) for _ in range(K)]
      + [pltpu.SemaphoreType.DMA for _ in range(2 * K)],
)(_posneg_body)


VPAD = 2432
TGB = 2048


def _tile_body(idx_ref, w_ref, out_ref, emb_ref):
    @pl.when(pl.program_id(1) == 0)
    def _():
        idx = idx_ref[0, 0, :]
        iota = lax.broadcasted_iota(jnp.int32, (TGB, VPAD), 1)
        oh = (idx[:, None] == iota).astype(jnp.float32)
        emb_ref[...] = jnp.dot(oh, w_ref[...],
                               preferred_element_type=jnp.float32)
    out_ref[...] = emb_ref[...]


_tile = pl.pallas_call(
    _tile_body,
    grid=(B // TGB, WALK),
    in_specs=[
        pl.BlockSpec((1, 1, TGB), lambda j, k: (j, 0, 0)),
        pl.BlockSpec((VPAD, E), lambda j, k: (0, 0)),
    ],
    out_specs=pl.BlockSpec((TGB, E), lambda j, k: (k * (B // TGB) + j, 0)),
    out_shape=jax.ShapeDtypeStruct((B * WALK, E), jnp.float32),
    scratch_shapes=[pltpu.VMEM((TGB, E), jnp.float32)],
)


def kernel(input_labels, pos_labels, neg_labels, W_in, W_out):
    pos_idx = pos_labels.reshape(NW, POS_CH, C).astype(jnp.int32)
    neg_idx = neg_labels.reshape(NW, NEG_CH, C).astype(jnp.int32)
    out_pos, out_neg = _posneg(pos_idx, neg_idx, W_out)
    tc_idx = input_labels.reshape(B // TGB, 1, TGB).astype(jnp.int32)
    w_in_pad = jnp.pad(W_in, ((0, VPAD - W_in.shape[0]), (0, 0)))
    out_in = _tile(tc_idx, w_in_pad)
    return out_in, out_pos, out_neg
